# baseline (speedup 1.0000x reference)
import functools

import jax
import jax.numpy as jnp
from jax import lax
from jax.experimental import pallas as pl
from jax.experimental.pallas import tpu as pltpu
from jax.experimental.pallas import tpu_sc as plsc

N_NODES = 20000
N_EDGES = 320000
X_DIM = 128
HIDDEN = 64
Y_DIM = 32

NC = 2
NS = 16
NW = NC * NS
CHUNK = 128
NCHUNK = 160
N0 = 56
N1 = NCHUNK - N0
NMAX = max(N0, N1)
TOT_CHUNKS = NS * NCHUNK
E_PAD = TOT_CHUNKS * CHUNK
DUMMY = N_NODES
ACC_ROWS = 20480
ZSLAB = ACC_ROWS // NS
OSLAB = N_NODES // NS

_MESH = plsc.VectorSubcoreMesh(core_axis_name="c", subcore_axis_name="s")


def _deg_kernel_body(dst_hbm, deg_out, idx_v, ones_v, stage_v, acc, sem):
    c = lax.axis_index("c")
    s = lax.axis_index("s")
    wid = c * NS + s

    def fill_ones(i, _):
        ones_v[pl.ds(i * 16, 16)] = jnp.ones((16,), jnp.float32)
        return 0

    lax.fori_loop(0, CHUNK // 16, fill_ones, 0)

    def fill_zero(i, _):
        stage_v[pl.ds(i * 16, 16)] = jnp.zeros((16,), jnp.float32)
        return 0

    lax.fori_loop(0, ZSLAB // 16, fill_zero, 0)

    pltpu.sync_copy(stage_v, acc.at[pl.ds(s * ZSLAB, ZSLAB)])
    plsc.subcore_barrier()

    nd = TOT_CHUNKS // NW
    pltpu.sync_copy(dst_hbm.at[pl.ds(wid * nd, nd)], idx_v)

    def edge_chunk(j, _):
        pltpu.sync_copy(ones_v, acc.at[idx_v.at[j]], add=True)
        return 0

    lax.fori_loop(0, nd, edge_chunk, 0)
    plsc.subcore_barrier()

    pltpu.sync_copy(acc.at[pl.ds(s * ZSLAB, ZSLAB)], stage_v)
    pltpu.sync_copy(stage_v, deg_out.at[c, pl.ds(s * ZSLAB, ZSLAB)])


_deg_kernel = pl.kernel(
    _deg_kernel_body,
    out_type=jax.ShapeDtypeStruct((NC, ACC_ROWS), jnp.float32),
    mesh=_MESH,
    scratch_types=[
        pltpu.VMEM((TOT_CHUNKS // NW, CHUNK), jnp.int32),
        pltpu.VMEM((CHUNK,), jnp.float32),
        pltpu.VMEM((ZSLAB,), jnp.float32),
        pltpu.VMEM_SHARED((ACC_ROWS,), jnp.float32),
        pltpu.SemaphoreType.DMA,
    ],
    compiler_params=pltpu.CompilerParams(use_tc_tiling_on_sc=False),
)


def _make_layer_kernel(width):
    def body(hs_hbm, src_hbm, dst_hbm, part_out, srcv, dstv, rows, stage, acc, sem0, sem1):
        c = lax.axis_index("c")
        s = lax.axis_index("s")
        wid = c * NS + s

        def fill_zero(i, _):
            for k in range(width // 16):
                rows[0, i, pl.ds(k * 16, 16)] = jnp.zeros((16,), jnp.float32)
            return 0

        lax.fori_loop(0, CHUNK, fill_zero, 0)

        def zslab(j, _):
            pltpu.sync_copy(rows.at[0], acc.at[pl.ds(s * ZSLAB + j * CHUNK, CHUNK)])
            return 0

        lax.fori_loop(0, ZSLAB // CHUNK, zslab, 0)

        base = jnp.where(c == 0, s * N0, NS * N0 + s * N1)
        npairs = jnp.where(c == 0, N0 // 2, N1 // 2)
        pltpu.sync_copy(src_hbm.at[pl.ds(base, NMAX)], srcv)
        pltpu.sync_copy(dst_hbm.at[pl.ds(base, NMAX)], dstv)
        plsc.subcore_barrier()

        pltpu.async_copy(hs_hbm.at[srcv.at[0]], rows.at[0], sem0)
        pltpu.async_copy(hs_hbm.at[srcv.at[1]], rows.at[1], sem1)

        def edge_pair(k, _):
            j0 = 2 * k
            j1 = j0 + 1
            pltpu.make_async_copy(hs_hbm.at[srcv.at[j0]], rows.at[0], sem0).wait()
            pltpu.sync_copy(rows.at[0], acc.at[dstv.at[j0]], add=True)

            @pl.when(k < npairs - 1)
            def _():
                pltpu.async_copy(hs_hbm.at[srcv.at[j0 + 2]], rows.at[0], sem0)

            pltpu.make_async_copy(hs_hbm.at[srcv.at[j1]], rows.at[1], sem1).wait()
            pltpu.sync_copy(rows.at[1], acc.at[dstv.at[j1]], add=True)

            @pl.when(k < npairs - 1)
            def _():
                pltpu.async_copy(hs_hbm.at[srcv.at[j1 + 2]], rows.at[1], sem1)

            return 0

        lax.fori_loop(0, npairs, edge_pair, 0)
        plsc.subcore_barrier()

        def wout(j, _):
            r0 = s * ZSLAB + j * (ZSLAB // 2)
            pltpu.sync_copy(acc.at[pl.ds(r0, ZSLAB // 2)], stage)
            pltpu.sync_copy(stage, part_out.at[c, pl.ds(r0, ZSLAB // 2)])
            return 0

        lax.fori_loop(0, 2, wout, 0)

    return pl.kernel(
        body,
        out_type=jax.ShapeDtypeStruct((NC, ACC_ROWS, width), jnp.float32),
        mesh=_MESH,
        scratch_types=[
            pltpu.VMEM((NMAX, CHUNK), jnp.int32),
            pltpu.VMEM((NMAX, CHUNK), jnp.int32),
            pltpu.VMEM((2, CHUNK, width), jnp.float32),
            pltpu.VMEM((ZSLAB // 2, width), jnp.float32),
            pltpu.VMEM_SHARED((ACC_ROWS, width), jnp.float32),
            pltpu.SemaphoreType.DMA,
            pltpu.SemaphoreType.DMA,
        ],
        compiler_params=pltpu.CompilerParams(use_tc_tiling_on_sc=False),
    )


_layer32 = _make_layer_kernel(Y_DIM)


def _dinv_body(deg_ref, dinv_ref):
    dinv_ref[...] = lax.rsqrt(deg_ref[0] + deg_ref[1] + 1.0)


def _mm1_body(x_ref, dinv_ref, w1_ref, hs_ref):
    h = lax.dot_general(
        x_ref[0], w1_ref[...],
        (((0,), (0,)), ((), ())),
        preferred_element_type=jnp.float32,
    )
    hs_ref[...] = h * dinv_ref[...]


def _mm2_body(pa_ref, pb_ref, hs1_ref, dinv_ref, b1_ref, w2_ref, hs2_ref):
    dinv = dinv_ref[...]
    agg = jnp.concatenate(
        [pa_ref[0] + pa_ref[1], pb_ref[0] + pb_ref[1]], axis=1
    ) + hs1_ref[...]
    pre = agg * dinv + b1_ref[...]
    a = jnp.maximum(pre, 0.0)
    h2 = lax.dot_general(
        a, w2_ref[...],
        (((1,), (0,)), ((), ())),
        preferred_element_type=jnp.float32,
    )
    hs2_ref[...] = h2 * dinv


def _fin_body(q_ref, hs2_ref, dinv_ref, b2_ref, out_ref):
    out_ref[...] = (q_ref[0] + q_ref[1] + hs2_ref[...]) * dinv_ref[...] + b2_ref[...]


def kernel(x_input, edge_index, W1, b1, W2, b2):
    pad = E_PAD - N_EDGES
    src = jnp.concatenate([edge_index[0], jnp.zeros((pad,), jnp.int32)])
    dst = jnp.concatenate([edge_index[1], jnp.full((pad,), DUMMY, jnp.int32)])
    src3 = src.reshape(TOT_CHUNKS, CHUNK)
    dst3 = dst.reshape(TOT_CHUNKS, CHUNK)

    degp = _deg_kernel(dst3)
    degp3 = degp.reshape(NC, ACC_ROWS, 1)

    half = N_NODES // 2
    hw = HIDDEN // 2
    rows = 2000

    dinv = pl.pallas_call(
        _dinv_body,
        grid=(N_NODES // rows,),
        in_specs=[pl.BlockSpec((NC, rows, 1), lambda j: (0, j, 0))],
        out_specs=pl.BlockSpec((rows, 1), lambda j: (j, 0)),
        out_shape=jax.ShapeDtypeStruct((N_NODES, 1), jnp.float32),
    )(degp3)

    hs1 = pl.pallas_call(
        _mm1_body,
        grid=(2,),
        in_specs=[
            pl.BlockSpec((1, X_DIM, half), lambda b: (b, 0, 0)),
            pl.BlockSpec((half, 1), lambda b: (b, 0)),
            pl.BlockSpec((X_DIM, HIDDEN), lambda b: (0, 0)),
        ],
        out_specs=pl.BlockSpec((half, HIDDEN), lambda b: (b, 0)),
        out_shape=jax.ShapeDtypeStruct((N_NODES, HIDDEN), jnp.float32),
    )(x_input, dinv, W1)

    hs1a = hs1[:, :hw]
    hs1b = hs1[:, hw:]
    parta = _layer32(hs1a, src3, dst3)
    partb = _layer32(hs1b, src3, dst3)

    hs2 = pl.pallas_call(
        _mm2_body,
        grid=(N_NODES // rows,),
        in_specs=[
            pl.BlockSpec((NC, rows, hw), lambda j: (0, j, 0)),
            pl.BlockSpec((NC, rows, hw), lambda j: (0, j, 0)),
            pl.BlockSpec((rows, HIDDEN), lambda j: (j, 0)),
            pl.BlockSpec((rows, 1), lambda j: (j, 0)),
            pl.BlockSpec((1, HIDDEN), lambda j: (0, 0)),
            pl.BlockSpec((HIDDEN, Y_DIM), lambda j: (0, 0)),
        ],
        out_specs=pl.BlockSpec((rows, Y_DIM), lambda j: (j, 0)),
        out_shape=jax.ShapeDtypeStruct((N_NODES, Y_DIM), jnp.float32),
    )(parta, partb, hs1, dinv, b1.reshape(1, HIDDEN), W2)

    part2 = _layer32(hs2, src3, dst3)

    out = pl.pallas_call(
        _fin_body,
        grid=(N_NODES // rows,),
        in_specs=[
            pl.BlockSpec((NC, rows, Y_DIM), lambda j: (0, j, 0)),
            pl.BlockSpec((rows, Y_DIM), lambda j: (j, 0)),
            pl.BlockSpec((rows, 1), lambda j: (j, 0)),
            pl.BlockSpec((1, Y_DIM), lambda j: (0, 0)),
        ],
        out_specs=pl.BlockSpec((rows, Y_DIM), lambda j: (j, 0)),
        out_shape=jax.ShapeDtypeStruct((N_NODES, Y_DIM), jnp.float32),
    )(part2, hs2, dinv, b2.reshape(1, Y_DIM))

    return out.reshape(x_input.shape[0], -1, Y_DIM).transpose(0, 2, 1)


# speedup vs baseline: 1.1035x; 1.1035x over previous
import functools

import jax
import jax.numpy as jnp
from jax import lax
from jax.experimental import pallas as pl
from jax.experimental.pallas import tpu as pltpu
from jax.experimental.pallas import tpu_sc as plsc

N_NODES = 20000
N_EDGES = 320000
X_DIM = 128
HIDDEN = 64
Y_DIM = 32

NC = 2
NS = 16
NW = NC * NS
CHUNK = 128
NCHUNK = 160
N0 = 104
N1 = NCHUNK - N0
NMAX = max(N0, N1)
TOT_CHUNKS = NS * NCHUNK
E_PAD = TOT_CHUNKS * CHUNK
DUMMY = N_NODES
ACC_ROWS = 20480
ZSLAB = ACC_ROWS // NS
OSLAB = N_NODES // NS

_MESH = plsc.VectorSubcoreMesh(core_axis_name="c", subcore_axis_name="s")


def _deg_kernel_body(dst_hbm, deg_out, idx_v, ones_v, stage_v, acc, sem):
    c = lax.axis_index("c")
    s = lax.axis_index("s")
    wid = c * NS + s

    def fill_ones(i, _):
        ones_v[pl.ds(i * 16, 16)] = jnp.ones((16,), jnp.float32)
        return 0

    lax.fori_loop(0, CHUNK // 16, fill_ones, 0)

    def fill_zero(i, _):
        stage_v[pl.ds(i * 16, 16)] = jnp.zeros((16,), jnp.float32)
        return 0

    lax.fori_loop(0, ZSLAB // 16, fill_zero, 0)

    pltpu.sync_copy(stage_v, acc.at[pl.ds(s * ZSLAB, ZSLAB)])
    plsc.subcore_barrier()

    nd = TOT_CHUNKS // NW
    pltpu.sync_copy(dst_hbm.at[pl.ds(wid * nd, nd)], idx_v)

    def edge_chunk(j, _):
        pltpu.sync_copy(ones_v, acc.at[idx_v.at[j]], add=True)
        return 0

    lax.fori_loop(0, nd, edge_chunk, 0)
    plsc.subcore_barrier()

    pltpu.sync_copy(acc.at[pl.ds(s * ZSLAB, ZSLAB)], stage_v)
    pltpu.sync_copy(stage_v, deg_out.at[c, pl.ds(s * ZSLAB, ZSLAB)])


_deg_kernel = pl.kernel(
    _deg_kernel_body,
    out_type=jax.ShapeDtypeStruct((NC, ACC_ROWS), jnp.float32),
    mesh=_MESH,
    scratch_types=[
        pltpu.VMEM((TOT_CHUNKS // NW, CHUNK), jnp.int32),
        pltpu.VMEM((CHUNK,), jnp.float32),
        pltpu.VMEM((ZSLAB,), jnp.float32),
        pltpu.VMEM_SHARED((ACC_ROWS,), jnp.float32),
        pltpu.SemaphoreType.DMA,
    ],
    compiler_params=pltpu.CompilerParams(use_tc_tiling_on_sc=False),
)


def _make_layer_kernel(width):
    def body(hs_hbm, src_hbm, dst_hbm, part_out, srcv, dstv, rows, stage, acc, sem0, sem1):
        c = lax.axis_index("c")
        s = lax.axis_index("s")
        wid = c * NS + s

        def fill_zero(i, _):
            for k in range(width // 16):
                rows[0, i, pl.ds(k * 16, 16)] = jnp.zeros((16,), jnp.float32)
            return 0

        lax.fori_loop(0, CHUNK, fill_zero, 0)

        def zslab(j, _):
            pltpu.sync_copy(rows.at[0], acc.at[pl.ds(s * ZSLAB + j * CHUNK, CHUNK)])
            return 0

        lax.fori_loop(0, ZSLAB // CHUNK, zslab, 0)

        base = jnp.where(c == 0, s * N0, NS * N0 + s * N1)
        npairs = jnp.where(c == 0, N0 // 2, N1 // 2)
        pltpu.sync_copy(src_hbm.at[pl.ds(base, NMAX)], srcv)
        pltpu.sync_copy(dst_hbm.at[pl.ds(base, NMAX)], dstv)
        plsc.subcore_barrier()

        pltpu.async_copy(hs_hbm.at[srcv.at[0]], rows.at[0], sem0)
        pltpu.async_copy(hs_hbm.at[srcv.at[1]], rows.at[1], sem1)

        def edge_pair(k, _):
            j0 = 2 * k
            j1 = j0 + 1
            pltpu.make_async_copy(hs_hbm.at[srcv.at[j0]], rows.at[0], sem0).wait()
            pltpu.sync_copy(rows.at[0], acc.at[dstv.at[j0]], add=True)

            @pl.when(k < npairs - 1)
            def _():
                pltpu.async_copy(hs_hbm.at[srcv.at[j0 + 2]], rows.at[0], sem0)

            pltpu.make_async_copy(hs_hbm.at[srcv.at[j1]], rows.at[1], sem1).wait()
            pltpu.sync_copy(rows.at[1], acc.at[dstv.at[j1]], add=True)

            @pl.when(k < npairs - 1)
            def _():
                pltpu.async_copy(hs_hbm.at[srcv.at[j1 + 2]], rows.at[1], sem1)

            return 0

        lax.fori_loop(0, npairs, edge_pair, 0)
        plsc.subcore_barrier()

        def wout(j, _):
            r0 = s * ZSLAB + j * (ZSLAB // 2)
            pltpu.sync_copy(acc.at[pl.ds(r0, ZSLAB // 2)], stage)
            pltpu.sync_copy(stage, part_out.at[c, pl.ds(r0, ZSLAB // 2)])
            return 0

        lax.fori_loop(0, 2, wout, 0)

    return pl.kernel(
        body,
        out_type=jax.ShapeDtypeStruct((NC, ACC_ROWS, width), jnp.float32),
        mesh=_MESH,
        scratch_types=[
            pltpu.VMEM((NMAX, CHUNK), jnp.int32),
            pltpu.VMEM((NMAX, CHUNK), jnp.int32),
            pltpu.VMEM((2, CHUNK, width), jnp.float32),
            pltpu.VMEM((ZSLAB // 2, width), jnp.float32),
            pltpu.VMEM_SHARED((ACC_ROWS, width), jnp.float32),
            pltpu.SemaphoreType.DMA,
            pltpu.SemaphoreType.DMA,
        ],
        compiler_params=pltpu.CompilerParams(use_tc_tiling_on_sc=False),
    )


_layer32 = _make_layer_kernel(Y_DIM)


def _dinv_body(deg_ref, dinv_ref):
    dinv_ref[...] = lax.rsqrt(deg_ref[0] + deg_ref[1] + 1.0)


def _mm1_body(x_ref, dinv_ref, w1_ref, hs_ref):
    h = lax.dot_general(
        x_ref[0], w1_ref[...],
        (((0,), (0,)), ((), ())),
        preferred_element_type=jnp.float32,
    )
    hs_ref[...] = h * dinv_ref[...]


def _mm2_body(pa_ref, pb_ref, hs1_ref, dinv_ref, b1_ref, w2_ref, hs2_ref):
    dinv = dinv_ref[...]
    agg = jnp.concatenate(
        [pa_ref[0] + pa_ref[1], pb_ref[0] + pb_ref[1]], axis=1
    ) + hs1_ref[...]
    pre = agg * dinv + b1_ref[...]
    a = jnp.maximum(pre, 0.0)
    h2 = lax.dot_general(
        a, w2_ref[...],
        (((1,), (0,)), ((), ())),
        preferred_element_type=jnp.float32,
    )
    hs2_ref[...] = h2 * dinv


def _fin_body(q_ref, hs2_ref, dinv_ref, b2_ref, out_ref):
    out_ref[...] = (q_ref[0] + q_ref[1] + hs2_ref[...]) * dinv_ref[...] + b2_ref[...]


def kernel(x_input, edge_index, W1, b1, W2, b2):
    pad = E_PAD - N_EDGES
    src = jnp.concatenate([edge_index[0], jnp.zeros((pad,), jnp.int32)])
    dst = jnp.concatenate([edge_index[1], jnp.full((pad,), DUMMY, jnp.int32)])
    src3 = src.reshape(TOT_CHUNKS, CHUNK)
    dst3 = dst.reshape(TOT_CHUNKS, CHUNK)

    degp = _deg_kernel(dst3)
    degp3 = degp.reshape(NC, ACC_ROWS, 1)

    half = N_NODES // 2
    hw = HIDDEN // 2
    rows = 2000

    dinv = pl.pallas_call(
        _dinv_body,
        grid=(N_NODES // rows,),
        in_specs=[pl.BlockSpec((NC, rows, 1), lambda j: (0, j, 0))],
        out_specs=pl.BlockSpec((rows, 1), lambda j: (j, 0)),
        out_shape=jax.ShapeDtypeStruct((N_NODES, 1), jnp.float32),
    )(degp3)

    hs1 = pl.pallas_call(
        _mm1_body,
        grid=(2,),
        in_specs=[
            pl.BlockSpec((1, X_DIM, half), lambda b: (b, 0, 0)),
            pl.BlockSpec((half, 1), lambda b: (b, 0)),
            pl.BlockSpec((X_DIM, HIDDEN), lambda b: (0, 0)),
        ],
        out_specs=pl.BlockSpec((half, HIDDEN), lambda b: (b, 0)),
        out_shape=jax.ShapeDtypeStruct((N_NODES, HIDDEN), jnp.float32),
    )(x_input, dinv, W1)

    hs1a = hs1[:, :hw]
    hs1b = hs1[:, hw:]
    parta = _layer32(hs1a, src3, dst3)
    partb = _layer32(hs1b, src3, dst3)

    hs2 = pl.pallas_call(
        _mm2_body,
        grid=(N_NODES // rows,),
        in_specs=[
            pl.BlockSpec((NC, rows, hw), lambda j: (0, j, 0)),
            pl.BlockSpec((NC, rows, hw), lambda j: (0, j, 0)),
            pl.BlockSpec((rows, HIDDEN), lambda j: (j, 0)),
            pl.BlockSpec((rows, 1), lambda j: (j, 0)),
            pl.BlockSpec((1, HIDDEN), lambda j: (0, 0)),
            pl.BlockSpec((HIDDEN, Y_DIM), lambda j: (0, 0)),
        ],
        out_specs=pl.BlockSpec((rows, Y_DIM), lambda j: (j, 0)),
        out_shape=jax.ShapeDtypeStruct((N_NODES, Y_DIM), jnp.float32),
    )(parta, partb, hs1, dinv, b1.reshape(1, HIDDEN), W2)

    part2 = _layer32(hs2, src3, dst3)

    out = pl.pallas_call(
        _fin_body,
        grid=(N_NODES // rows,),
        in_specs=[
            pl.BlockSpec((NC, rows, Y_DIM), lambda j: (0, j, 0)),
            pl.BlockSpec((rows, Y_DIM), lambda j: (j, 0)),
            pl.BlockSpec((rows, 1), lambda j: (j, 0)),
            pl.BlockSpec((1, Y_DIM), lambda j: (0, 0)),
        ],
        out_specs=pl.BlockSpec((rows, Y_DIM), lambda j: (j, 0)),
        out_shape=jax.ShapeDtypeStruct((N_NODES, Y_DIM), jnp.float32),
    )(part2, hs2, dinv, b2.reshape(1, Y_DIM))

    return out.reshape(x_input.shape[0], -1, Y_DIM).transpose(0, 2, 1)


# speedup vs baseline: 1.1353x; 1.0288x over previous
import functools

import jax
import jax.numpy as jnp
from jax import lax
from jax.experimental import pallas as pl
from jax.experimental.pallas import tpu as pltpu
from jax.experimental.pallas import tpu_sc as plsc

N_NODES = 20000
N_EDGES = 320000
X_DIM = 128
HIDDEN = 64
Y_DIM = 32

NC = 2
NS = 16
NW = NC * NS
CHUNK = 128
NCHUNK = 160
N0 = 134
N1 = NCHUNK - N0
NMAX = max(N0, N1)
TOT_CHUNKS = NS * NCHUNK
E_PAD = TOT_CHUNKS * CHUNK
DUMMY = N_NODES
ACC_ROWS = 20480
ZSLAB = ACC_ROWS // NS
OSLAB = N_NODES // NS

_MESH = plsc.VectorSubcoreMesh(core_axis_name="c", subcore_axis_name="s")


def _deg_kernel_body(dst_hbm, deg_out, idx_v, ones_v, stage_v, acc, sem):
    c = lax.axis_index("c")
    s = lax.axis_index("s")
    wid = c * NS + s

    def fill_ones(i, _):
        ones_v[pl.ds(i * 16, 16)] = jnp.ones((16,), jnp.float32)
        return 0

    lax.fori_loop(0, CHUNK // 16, fill_ones, 0)

    def fill_zero(i, _):
        stage_v[pl.ds(i * 16, 16)] = jnp.zeros((16,), jnp.float32)
        return 0

    lax.fori_loop(0, ZSLAB // 16, fill_zero, 0)

    pltpu.sync_copy(stage_v, acc.at[pl.ds(s * ZSLAB, ZSLAB)])
    plsc.subcore_barrier()

    nd = TOT_CHUNKS // NW
    pltpu.sync_copy(dst_hbm.at[pl.ds(wid * nd, nd)], idx_v)

    def edge_chunk(j, _):
        pltpu.sync_copy(ones_v, acc.at[idx_v.at[j]], add=True)
        return 0

    lax.fori_loop(0, nd, edge_chunk, 0)
    plsc.subcore_barrier()

    pltpu.sync_copy(acc.at[pl.ds(s * ZSLAB, ZSLAB)], stage_v)
    pltpu.sync_copy(stage_v, deg_out.at[c, pl.ds(s * ZSLAB, ZSLAB)])


_deg_kernel = pl.kernel(
    _deg_kernel_body,
    out_type=jax.ShapeDtypeStruct((NC, ACC_ROWS), jnp.float32),
    mesh=_MESH,
    scratch_types=[
        pltpu.VMEM((TOT_CHUNKS // NW, CHUNK), jnp.int32),
        pltpu.VMEM((CHUNK,), jnp.float32),
        pltpu.VMEM((ZSLAB,), jnp.float32),
        pltpu.VMEM_SHARED((ACC_ROWS,), jnp.float32),
        pltpu.SemaphoreType.DMA,
    ],
    compiler_params=pltpu.CompilerParams(use_tc_tiling_on_sc=False),
)


def _make_layer_kernel(width):
    def body(hs_hbm, src_hbm, dst_hbm, part_out, srcv, dstv, rows, stage, acc, sem0, sem1):
        c = lax.axis_index("c")
        s = lax.axis_index("s")
        wid = c * NS + s

        def fill_zero(i, _):
            for k in range(width // 16):
                rows[0, i, pl.ds(k * 16, 16)] = jnp.zeros((16,), jnp.float32)
            return 0

        lax.fori_loop(0, CHUNK, fill_zero, 0)

        def zslab(j, _):
            pltpu.sync_copy(rows.at[0], acc.at[pl.ds(s * ZSLAB + j * CHUNK, CHUNK)])
            return 0

        lax.fori_loop(0, ZSLAB // CHUNK, zslab, 0)

        base = jnp.where(c == 0, s * N0, NS * N0 + s * N1)
        npairs = jnp.where(c == 0, N0 // 2, N1 // 2)
        pltpu.sync_copy(src_hbm.at[pl.ds(base, NMAX)], srcv)
        pltpu.sync_copy(dst_hbm.at[pl.ds(base, NMAX)], dstv)
        plsc.subcore_barrier()

        pltpu.async_copy(hs_hbm.at[srcv.at[0]], rows.at[0], sem0)
        pltpu.async_copy(hs_hbm.at[srcv.at[1]], rows.at[1], sem1)

        def edge_pair(k, _):
            j0 = 2 * k
            j1 = j0 + 1
            pltpu.make_async_copy(hs_hbm.at[srcv.at[j0]], rows.at[0], sem0).wait()
            pltpu.sync_copy(rows.at[0], acc.at[dstv.at[j0]], add=True)

            @pl.when(k < npairs - 1)
            def _():
                pltpu.async_copy(hs_hbm.at[srcv.at[j0 + 2]], rows.at[0], sem0)

            pltpu.make_async_copy(hs_hbm.at[srcv.at[j1]], rows.at[1], sem1).wait()
            pltpu.sync_copy(rows.at[1], acc.at[dstv.at[j1]], add=True)

            @pl.when(k < npairs - 1)
            def _():
                pltpu.async_copy(hs_hbm.at[srcv.at[j1 + 2]], rows.at[1], sem1)

            return 0

        lax.fori_loop(0, npairs, edge_pair, 0)
        plsc.subcore_barrier()

        def wout(j, _):
            r0 = s * ZSLAB + j * (ZSLAB // 2)
            pltpu.sync_copy(acc.at[pl.ds(r0, ZSLAB // 2)], stage)
            pltpu.sync_copy(stage, part_out.at[c, pl.ds(r0, ZSLAB // 2)])
            return 0

        lax.fori_loop(0, 2, wout, 0)

    return pl.kernel(
        body,
        out_type=jax.ShapeDtypeStruct((NC, ACC_ROWS, width), jnp.float32),
        mesh=_MESH,
        scratch_types=[
            pltpu.VMEM((NMAX, CHUNK), jnp.int32),
            pltpu.VMEM((NMAX, CHUNK), jnp.int32),
            pltpu.VMEM((2, CHUNK, width), jnp.float32),
            pltpu.VMEM((ZSLAB // 2, width), jnp.float32),
            pltpu.VMEM_SHARED((ACC_ROWS, width), jnp.float32),
            pltpu.SemaphoreType.DMA,
            pltpu.SemaphoreType.DMA,
        ],
        compiler_params=pltpu.CompilerParams(use_tc_tiling_on_sc=False),
    )


_layer32 = _make_layer_kernel(Y_DIM)


def _dinv_body(deg_ref, dinv_ref):
    dinv_ref[...] = lax.rsqrt(deg_ref[0] + deg_ref[1] + 1.0)


def _mm1_body(x_ref, dinv_ref, w1_ref, hs_ref):
    h = lax.dot_general(
        x_ref[0], w1_ref[...],
        (((0,), (0,)), ((), ())),
        preferred_element_type=jnp.float32,
    )
    hs_ref[...] = h * dinv_ref[...]


def _mm2_body(pa_ref, pb_ref, hs1_ref, dinv_ref, b1_ref, w2_ref, hs2_ref):
    dinv = dinv_ref[...]
    agg = jnp.concatenate(
        [pa_ref[0] + pa_ref[1], pb_ref[0] + pb_ref[1]], axis=1
    ) + hs1_ref[...]
    pre = agg * dinv + b1_ref[...]
    a = jnp.maximum(pre, 0.0)
    h2 = lax.dot_general(
        a, w2_ref[...],
        (((1,), (0,)), ((), ())),
        preferred_element_type=jnp.float32,
    )
    hs2_ref[...] = h2 * dinv


def _fin_body(q_ref, hs2_ref, dinv_ref, b2_ref, out_ref):
    s = (q_ref[0] + q_ref[1] + hs2_ref[...]) * dinv_ref[...] + b2_ref[...]
    out_ref[0] = s.T


def kernel(x_input, edge_index, W1, b1, W2, b2):
    pad = E_PAD - N_EDGES
    src = jnp.concatenate([edge_index[0], jnp.zeros((pad,), jnp.int32)])
    dst = jnp.concatenate([edge_index[1], jnp.full((pad,), DUMMY, jnp.int32)])
    src3 = src.reshape(TOT_CHUNKS, CHUNK)
    dst3 = dst.reshape(TOT_CHUNKS, CHUNK)

    degp = _deg_kernel(dst3)
    degp3 = degp.reshape(NC, ACC_ROWS, 1)

    half = N_NODES // 2
    hw = HIDDEN // 2
    rows = 2000

    dinv = pl.pallas_call(
        _dinv_body,
        grid=(N_NODES // rows,),
        in_specs=[pl.BlockSpec((NC, rows, 1), lambda j: (0, j, 0))],
        out_specs=pl.BlockSpec((rows, 1), lambda j: (j, 0)),
        out_shape=jax.ShapeDtypeStruct((N_NODES, 1), jnp.float32),
    )(degp3)

    hs1 = pl.pallas_call(
        _mm1_body,
        grid=(2,),
        in_specs=[
            pl.BlockSpec((1, X_DIM, half), lambda b: (b, 0, 0)),
            pl.BlockSpec((half, 1), lambda b: (b, 0)),
            pl.BlockSpec((X_DIM, HIDDEN), lambda b: (0, 0)),
        ],
        out_specs=pl.BlockSpec((half, HIDDEN), lambda b: (b, 0)),
        out_shape=jax.ShapeDtypeStruct((N_NODES, HIDDEN), jnp.float32),
    )(x_input, dinv, W1)

    hs1a = hs1[:, :hw]
    hs1b = hs1[:, hw:]
    parta = _layer32(hs1a, src3, dst3)
    partb = _layer32(hs1b, src3, dst3)

    hs2 = pl.pallas_call(
        _mm2_body,
        grid=(N_NODES // rows,),
        in_specs=[
            pl.BlockSpec((NC, rows, hw), lambda j: (0, j, 0)),
            pl.BlockSpec((NC, rows, hw), lambda j: (0, j, 0)),
            pl.BlockSpec((rows, HIDDEN), lambda j: (j, 0)),
            pl.BlockSpec((rows, 1), lambda j: (j, 0)),
            pl.BlockSpec((1, HIDDEN), lambda j: (0, 0)),
            pl.BlockSpec((HIDDEN, Y_DIM), lambda j: (0, 0)),
        ],
        out_specs=pl.BlockSpec((rows, Y_DIM), lambda j: (j, 0)),
        out_shape=jax.ShapeDtypeStruct((N_NODES, Y_DIM), jnp.float32),
    )(parta, partb, hs1, dinv, b1.reshape(1, HIDDEN), W2)

    part2 = _layer32(hs2, src3, dst3)

    out = pl.pallas_call(
        _fin_body,
        grid=(2,),
        in_specs=[
            pl.BlockSpec((NC, half, Y_DIM), lambda b: (0, b, 0)),
            pl.BlockSpec((half, Y_DIM), lambda b: (b, 0)),
            pl.BlockSpec((half, 1), lambda b: (b, 0)),
            pl.BlockSpec((1, Y_DIM), lambda b: (0, 0)),
        ],
        out_specs=pl.BlockSpec((1, Y_DIM, half), lambda b: (b, 0, 0)),
        out_shape=jax.ShapeDtypeStruct((2, Y_DIM, half), jnp.float32),
    )(part2, hs2, dinv, b2.reshape(1, Y_DIM))

    return out
